# Initial kernel scaffold; baseline (speedup 1.0000x reference)
import jax
import jax.numpy as jnp
from jax.experimental import pallas as pl


def kernel(z, codebooks):
    raise NotImplementedError("write your pallas kernel here")


# speedup vs baseline: 8.1653x; 8.1653x over previous
import functools

import jax
import jax.numpy as jnp
from jax.experimental import pallas as pl

EMBED_DIM = 256
NUM_CODES = 1024
NUM_SUB = 16
DS = EMBED_DIM // NUM_SUB


def _vq_body(z_ref, cb_ref, zq_ref, idx_ref, loss_ref):
    zb = z_ref[0, 0]
    cb = cb_ref[0]
    m = cb.shape[0]
    hw = zb.shape[1]

    c2 = jnp.sum(cb * cb, axis=1, keepdims=True)
    cross = jax.lax.dot_general(cb, zb, (((1,), (0,)), ((), ())))
    score = c2 - 2.0 * cross
    minv = jnp.min(score, axis=0, keepdims=True)
    iota_m = jax.lax.broadcasted_iota(jnp.int32, (m, hw), 0)
    idx = jnp.min(jnp.where(score == minv, iota_m, m), axis=0,
                  keepdims=True)
    idx_ref[0, 0] = idx

    onehot = (iota_m == idx).astype(jnp.float32)
    zq = jax.lax.dot_general(cb, onehot, (((0,), (0,)), ((), ())))
    zq_ref[0, 0] = zq

    r = zq - zb
    part = jnp.sum(r * r).reshape(1, 1)
    is_first = (pl.program_id(0) == 0) & (pl.program_id(1) == 0)

    @pl.when(is_first)
    def _():
        loss_ref[:, :] = part

    @pl.when(jnp.logical_not(is_first))
    def _():
        loss_ref[:, :] += part


@functools.partial(jax.jit, static_argnames=())
def kernel(z, codebooks):
    B, D, H, W = z.shape
    N, M, ds = codebooks.shape
    HW = H * W
    T = B * HW
    z4 = z.reshape(B, N, ds, HW)

    zq4, idx, loss_acc = pl.pallas_call(
        _vq_body,
        grid=(N, B),
        in_specs=[
            pl.BlockSpec((1, 1, ds, HW), lambda n, b: (b, n, 0, 0)),
            pl.BlockSpec((1, M, ds), lambda n, b: (n, 0, 0)),
        ],
        out_specs=[
            pl.BlockSpec((1, 1, ds, HW), lambda n, b: (b, n, 0, 0)),
            pl.BlockSpec((1, 1, 1, HW), lambda n, b: (n, b, 0, 0)),
            pl.BlockSpec((1, 1), lambda n, b: (0, 0)),
        ],
        out_shape=[
            jax.ShapeDtypeStruct((B, N, ds, HW), jnp.float32),
            jax.ShapeDtypeStruct((N, B, 1, HW), jnp.int32),
            jax.ShapeDtypeStruct((1, 1), jnp.float32),
        ],
    )(z4, codebooks)

    z_q_out = zq4.reshape(B, D, H, W)
    loss = loss_acc[0, 0] / jnp.float32(N * T * ds)
    indices = jnp.transpose(idx.reshape(N, T), (1, 0))
    return (z_q_out, loss, loss, indices)


# speedup vs baseline: 8.2227x; 1.0070x over previous
import functools

import jax
import jax.numpy as jnp
from jax.experimental import pallas as pl
from jax.experimental.pallas import tpu as pltpu

EMBED_DIM = 256
NUM_CODES = 1024
NUM_SUB = 16
DS = EMBED_DIM // NUM_SUB


def _vq_body(z_ref, cb_ref, zq_ref, idx_ref, loss_ref):
    zb = z_ref[0, 0]
    cb = cb_ref[0]
    m = cb.shape[0]
    hw = zb.shape[1]

    c2 = jnp.sum(cb * cb, axis=1, keepdims=True)
    cross = jax.lax.dot_general(cb, zb, (((1,), (0,)), ((), ())))
    score = c2 - 2.0 * cross
    minv = jnp.min(score, axis=0, keepdims=True)
    iota_m = jax.lax.broadcasted_iota(jnp.int32, (m, hw), 0)
    idx = jnp.min(jnp.where(score == minv, iota_m, m), axis=0,
                  keepdims=True)
    idx_ref[0, 0] = idx

    onehot = (iota_m == idx).astype(jnp.float32)
    zq = jax.lax.dot_general(cb, onehot, (((0,), (0,)), ((), ())))
    zq_ref[0, 0] = zq

    r = zq - zb
    loss_ref[0, 0] = jnp.sum(r * r).reshape(1, 1)


@functools.partial(jax.jit, static_argnames=())
def kernel(z, codebooks):
    B, D, H, W = z.shape
    N, M, ds = codebooks.shape
    HW = H * W
    T = B * HW
    z4 = z.reshape(B, N, ds, HW)

    zq4, idx, loss_acc = pl.pallas_call(
        _vq_body,
        grid=(N, B),
        in_specs=[
            pl.BlockSpec((1, 1, ds, HW), lambda n, b: (b, n, 0, 0)),
            pl.BlockSpec((1, M, ds), lambda n, b: (n, 0, 0)),
        ],
        out_specs=[
            pl.BlockSpec((1, 1, ds, HW), lambda n, b: (b, n, 0, 0)),
            pl.BlockSpec((1, 1, 1, HW), lambda n, b: (n, b, 0, 0)),
            pl.BlockSpec((1, 1, 1, 1), lambda n, b: (n, b, 0, 0)),
        ],
        out_shape=[
            jax.ShapeDtypeStruct((B, N, ds, HW), jnp.float32),
            jax.ShapeDtypeStruct((N, B, 1, HW), jnp.int32),
            jax.ShapeDtypeStruct((N, B, 1, 1), jnp.float32),
        ],
        compiler_params=pltpu.CompilerParams(
            dimension_semantics=("parallel", "parallel"),
        ),
    )(z4, codebooks)

    z_q_out = zq4.reshape(B, D, H, W)
    loss = jnp.sum(loss_acc) / jnp.float32(N * T * ds)
    indices = jnp.transpose(idx.reshape(N, T), (1, 0))
    return (z_q_out, loss, loss, indices)


# speedup vs baseline: 12.0716x; 1.4681x over previous
import functools

import jax
import jax.numpy as jnp
from jax.experimental import pallas as pl
from jax.experimental.pallas import tpu as pltpu

EMBED_DIM = 256
NUM_CODES = 1024
NUM_SUB = 16
DS = EMBED_DIM // NUM_SUB


def _vq_body(z_ref, cb_ref, zq_ref, idx_ref, loss_ref):
    cb = cb_ref[0]
    m = cb.shape[0]
    bb = z_ref.shape[0]
    hw = z_ref.shape[3]

    c2 = jnp.sum(cb * cb, axis=1, keepdims=True)
    iota_m = jax.lax.broadcasted_iota(jnp.int32, (m, hw), 0)
    part = jnp.zeros((1, 1), jnp.float32)
    for j in range(bb):
        zb = z_ref[j, 0]
        cross = jax.lax.dot_general(cb, zb, (((1,), (0,)), ((), ())))
        score = c2 - 2.0 * cross
        idx = jnp.argmin(score, axis=0).reshape(1, hw).astype(jnp.int32)
        idx_ref[0, j] = idx
        onehot = (iota_m == idx).astype(jnp.float32)
        zq = jax.lax.dot_general(cb, onehot, (((0,), (0,)), ((), ())))
        zq_ref[j, 0] = zq
        r = zq - zb
        part = part + jnp.sum(r * r).reshape(1, 1)
    loss_ref[0, 0] = part


@functools.partial(jax.jit, static_argnames=())
def kernel(z, codebooks):
    B, D, H, W = z.shape
    N, M, ds = codebooks.shape
    HW = H * W
    T = B * HW
    z4 = z.reshape(B, N, ds, HW)

    BB = 16
    zq4, idx, loss_acc = pl.pallas_call(
        _vq_body,
        grid=(N, B // BB),
        in_specs=[
            pl.BlockSpec((BB, 1, ds, HW), lambda n, b: (b, n, 0, 0)),
            pl.BlockSpec((1, M, ds), lambda n, b: (n, 0, 0)),
        ],
        out_specs=[
            pl.BlockSpec((BB, 1, ds, HW), lambda n, b: (b, n, 0, 0)),
            pl.BlockSpec((1, BB, 1, HW), lambda n, b: (n, b, 0, 0)),
            pl.BlockSpec((1, 1, 1, 1), lambda n, b: (n, b, 0, 0)),
        ],
        out_shape=[
            jax.ShapeDtypeStruct((B, N, ds, HW), jnp.float32),
            jax.ShapeDtypeStruct((N, B, 1, HW), jnp.int32),
            jax.ShapeDtypeStruct((N, B // BB, 1, 1), jnp.float32),
        ],
        compiler_params=pltpu.CompilerParams(
            dimension_semantics=("parallel", "parallel"),
        ),
    )(z4, codebooks)

    z_q_out = zq4.reshape(B, D, H, W)
    loss = jnp.sum(loss_acc) / jnp.float32(N * T * ds)
    indices = jnp.transpose(idx.reshape(N, T), (1, 0))
    return (z_q_out, loss, loss, indices)
